# baseline (speedup 1.0000x reference)
import functools

import jax
import jax.numpy as jnp
from jax import lax
from jax.experimental import pallas as pl
from jax.experimental.pallas import tpu as pltpu
from jax.experimental.pallas import tpu_sc as plsc

D = 64
B = 16384
NC = 2
NS = 16
L = 16
NW = NC * NS
BPW = B // NW
CHUNK = 128
NCHUNK = BPW // CHUNK

_mesh = plsc.VectorSubcoreMesh(core_axis_name="c", subcore_axis_name="s")


@functools.partial(
    pl.kernel,
    out_type=jax.ShapeDtypeStruct((B,), jnp.float32),
    mesh=_mesh,
    compiler_params=pltpu.CompilerParams(
        needs_layout_passes=False, use_tc_tiling_on_sc=False
    ),
    scratch_types=[
        pltpu.VMEM((NCHUNK, CHUNK), jnp.int32),
        pltpu.VMEM((NCHUNK, CHUNK), jnp.int32),
        pltpu.VMEM((BPW, D), jnp.float32),
        pltpu.VMEM((BPW, D), jnp.float32),
        pltpu.VMEM((BPW,), jnp.float32),
        pltpu.SemaphoreType.DMA,
        pltpu.SemaphoreType.DMA,
    ],
)
def _sc_dot_kernel(ti_hbm, cj_hbm, emb_hbm, out_hbm,
                   idx_t, idx_c, trows, crows, outv, sem_t, sem_c):
    wid = lax.axis_index("s") * NC + lax.axis_index("c")
    base = wid * BPW
    row0 = wid * NCHUNK

    pltpu.sync_copy(ti_hbm.at[pl.ds(row0, NCHUNK)], idx_t)
    pltpu.sync_copy(cj_hbm.at[pl.ds(row0, NCHUNK)], idx_c)

    copies = []
    for j in range(NCHUNK):
        copies.append(pltpu.async_copy(
            emb_hbm.at[idx_t.at[j]], trows.at[pl.ds(j * CHUNK, CHUNK)], sem_t))
        copies.append(pltpu.async_copy(
            emb_hbm.at[idx_c.at[j]], crows.at[pl.ds(j * CHUNK, CHUNK)], sem_c))
    for cp in copies:
        cp.wait()

    lanes = lax.iota(jnp.int32, L)

    def group_body(g, _):
        rows = g * L + lanes

        def dstep(d, acc):
            cols = jnp.full((L,), d, jnp.int32)
            tv = plsc.load_gather(trows, [rows, cols])
            cv = plsc.load_gather(crows, [rows, cols])
            return acc + tv * cv

        acc = lax.fori_loop(0, D, dstep, jnp.zeros((L,), jnp.float32))
        outv[pl.ds(g * L, L)] = 1.0 / (1.0 + jnp.exp(-acc))
        return 0

    lax.fori_loop(0, BPW // L, group_body, 0)

    pltpu.sync_copy(outv, out_hbm.at[pl.ds(base, BPW)])


def kernel(target_i, context_j, label, emb):
    ti = target_i.reshape(B // CHUNK, CHUNK)
    cj = context_j.reshape(B // CHUNK, CHUNK)
    out = _sc_dot_kernel(ti, cj, emb)
    return (out, label.astype(jnp.float32))
